# baseline (speedup 1.0000x reference)
import jax
import jax.numpy as jnp
from jax import lax
from jax.experimental import pallas as pl
from jax.experimental.pallas import tpu as pltpu
from jax.experimental.pallas import tpu_sc as plsc

NC, NS, L = 2, 16, 16
TOTAL = 8 * 384 * 384
PER_T = TOTAL // NS
CH = 2048
NCH = PER_T // CH
NBIN = 1024
NBINH = 1040
HISTW = L * NBINH
U = 8

_mesh = plsc.VectorSubcoreMesh(
    core_axis_name="c", subcore_axis_name="s", num_cores=NC, num_subcores=NS)
_params = pltpu.CompilerParams(needs_layout_passes=False)


def _fused_body(lab2, pre2, mm, cnt1, sum1, h2,
                st_lab0, st_lab1, st_pre0, st_pre1, st_msk0, st_msk1,
                vres, h_cnt, h_sum, red, redglob, comb0, comb1,
                semi0, semi1):
    cid = lax.axis_index("c")
    sid = lax.axis_index("s")
    ti = cid * NS + sid
    tbase = cid * TOTAL + sid * PER_T
    mbase = sid * PER_T
    lane_base = lax.iota(jnp.int32, L) * NBINH
    zeros = jnp.zeros((L,), jnp.float32)
    ones = jnp.ones((L,), jnp.float32)
    posbin = jnp.full((L,), NBIN, jnp.int32)
    semi = (semi0, semi1)
    st_lab = (st_lab0, st_lab1)
    st_pre = (st_pre0, st_pre1)
    st_msk = (st_msk0, st_msk1)
    combs = (comb0, comb1)

    def zero_hists():
        def zb(j, _):
            for u in range(8):
                h_cnt[pl.ds(j * 8 * L + u * L, L)] = zeros
                h_sum[pl.ds(j * 8 * L + u * L, L)] = zeros
            return 0
        lax.fori_loop(0, HISTW // (8 * L), zb, 0)

    def reduce_hist(h):
        def rb(c, _):
            acc = zeros
            for l in range(L):
                acc = acc + h[pl.ds(l * NBINH + c * L, L)]
            red[pl.ds(c * L, L)] = acc
            return 0
        lax.fori_loop(0, NBINH // L, rb, 0)

    zero_hists()
    in_h = {}

    def issue_in(c):
        buf = c & 1
        in_h[c] = [
            pltpu.async_copy(lab2.at[pl.ds(tbase + c * CH, CH)], st_lab[buf], semi[buf]),
            pltpu.async_copy(pre2.at[pl.ds(tbase + c * CH, CH)], st_pre[buf], semi[buf]),
            pltpu.async_copy(mm.at[pl.ds(mbase + c * CH, CH)], st_msk[buf], semi[buf]),
        ]

    issue_in(0)
    for c in range(NCH):
        buf = c & 1
        if c + 1 < NCH:
            issue_in(c + 1)
        for h in in_h.pop(c):
            h.wait()
        labr = st_lab[buf]
        prer = st_pre[buf]
        mskr = st_msk[buf]

        def vec(i, _):
            ss = [pl.ds(i * U * L + u * L, L) for u in range(U)]
            vs = [pl.ds(i * U * L + u * L + c * CH, L) for u in range(U)]
            labs = [labr[s] for s in ss]
            pres = [prer[s] for s in ss]
            msks = [mskr[s] for s in ss]
            dds = [pres[u] - labs[u] for u in range(U)]
            sqs = [dds[u] * dds[u] for u in range(U)]
            plss = [sqs[u] * msks[u] for u in range(U)]
            poss = [labs[u] > 0.1 for u in range(U)]
            vvs = [jnp.where(poss[u], zeros, plss[u]) for u in range(U)]
            bits = [plsc.bitcast(vvs[u], jnp.int32) >> 21 for u in range(U)]
            addrs = [lane_base + jnp.where(poss[u], posbin, bits[u])
                     for u in range(U)]
            for u in range(U):
                vres[vs[u]] = vvs[u]
            for u in range(U):
                plsc.addupdate_scatter(h_sum, [addrs[u]], plss[u])
                plsc.addupdate_scatter(h_cnt, [addrs[u]], ones)
            return 0

        lax.fori_loop(0, CH // (U * L), vec, 0)

    reduce_hist(h_cnt)
    pltpu.sync_copy(red, cnt1.at[pl.ds(ti * NBINH, NBINH)])
    reduce_hist(h_sum)
    pltpu.sync_copy(red, sum1.at[pl.ds(ti * NBINH, NBINH)])

    zero_hists()

    def zg(g, _):
        redglob[pl.ds(g * L, L)] = zeros
        return 0
    lax.fori_loop(0, NBINH // L, zg, 0)

    plsc.subcore_barrier()

    cb = cid * (NS * NBINH)
    rb_h = {}

    def issue_rb(j):
        rb_h[j] = pltpu.async_copy(
            cnt1.at[pl.ds(cb + j * 4 * NBINH, 4 * NBINH)], combs[j & 1],
            semi[j & 1])

    issue_rb(0)
    for j in range(NS // 4):
        if j + 1 < NS // 4:
            issue_rb(j + 1)
        rb_h.pop(j).wait()
        cmb = combs[j & 1]

        def rbody(g, _):
            s = pl.ds(g * L, L)
            acc = redglob[s]
            for r in range(4):
                acc = acc + cmb[pl.ds(g * L + r * NBINH, L)]
            redglob[s] = acc
            return 0
        lax.fori_loop(0, NBINH // L, rbody, 0)

    grpP = redglob[pl.ds(NBIN, L)]
    kk = 3.0 * jnp.sum(grpP)

    def tb(g, tot):
        return tot + jnp.sum(redglob[pl.ds(g * L, L)])
    total_c = lax.fori_loop(0, NBIN // L, tb, jnp.float32(0.0))

    def sb(g, carry):
        cc, b1c, fnd = carry
        grp = redglob[pl.ds(g * L, L)]
        cum = plsc.cumsum(grp)
        hit = (total_c - (cc + cum)) < kk
        popc = plsc.all_reduce_population_count(hit)
        cand = g * L + (L - popc)
        b1n = jnp.where(fnd, b1c, jnp.where(popc > 0, cand, b1c))
        fnd = jnp.logical_or(fnd, popc > 0)
        return cc + jnp.sum(grp), b1n, fnd

    _, b1v, _ = lax.fori_loop(
        0, NBIN // L, sb,
        (jnp.float32(0.0), jnp.zeros((L,), jnp.int32),
         jnp.zeros((L,), jnp.bool_)))

    def vec2(i, _):
        ss = [pl.ds(i * U * L + u * L, L) for u in range(U)]
        vvs = [vres[s] for s in ss]
        bits = [plsc.bitcast(vvs[u], jnp.int32) for u in range(U)]
        matches = [(bits[u] >> 21) == b1v for u in range(U)]
        addrs = [lane_base + ((bits[u] >> 11) & 0x3FF) for u in range(U)]
        for u in range(U):
            plsc.addupdate_scatter(h_sum, [addrs[u]], vvs[u], mask=matches[u])
            plsc.addupdate_scatter(h_cnt, [addrs[u]], ones, mask=matches[u])
        return 0

    lax.fori_loop(0, PER_T // (U * L), vec2, 0)

    reduce_hist(h_cnt)
    pltpu.sync_copy(red, h2.at[pl.ds(ti * 2 * NBINH, NBINH)])
    reduce_hist(h_sum)
    pltpu.sync_copy(red, h2.at[pl.ds(ti * 2 * NBINH + NBINH, NBINH)])


_fused = pl.kernel(
    _fused_body,
    out_type=[
        jax.ShapeDtypeStruct((NC * NS * NBINH,), jnp.float32),
        jax.ShapeDtypeStruct((NC * NS * NBINH,), jnp.float32),
        jax.ShapeDtypeStruct((NC * NS * 2 * NBINH,), jnp.float32),
    ],
    mesh=_mesh,
    compiler_params=_params,
    scratch_types=[
        pltpu.VMEM((CH,), jnp.float32),
        pltpu.VMEM((CH,), jnp.float32),
        pltpu.VMEM((CH,), jnp.float32),
        pltpu.VMEM((CH,), jnp.float32),
        pltpu.VMEM((CH,), jnp.float32),
        pltpu.VMEM((CH,), jnp.float32),
        pltpu.VMEM((PER_T,), jnp.float32),
        pltpu.VMEM((HISTW,), jnp.float32),
        pltpu.VMEM((HISTW,), jnp.float32),
        pltpu.VMEM((NBINH,), jnp.float32),
        pltpu.VMEM((NBINH,), jnp.float32),
        pltpu.VMEM((4 * NBINH,), jnp.float32),
        pltpu.VMEM((4 * NBINH,), jnp.float32),
        pltpu.SemaphoreType.DMA,
        pltpu.SemaphoreType.DMA,
    ],
)


def _select(cnt, s, k):
    cc = jnp.cumsum(cnt)
    cs = jnp.cumsum(s)
    above_c = cc[-1] - cc
    above_s = cs[-1] - cs
    hit = above_c < k
    b = jnp.where(jnp.any(hit), jnp.argmax(hit), 0).astype(jnp.int32)
    return b, k - above_c[b], above_s[b]


def kernel(region_scores_label, affinity_socres_label, region_scores_pre,
           affinity_scores_pre, mask):
    rl = region_scores_label.reshape(-1)
    al = affinity_socres_label.reshape(-1)
    rp = region_scores_pre.reshape(-1)
    ap = affinity_scores_pre.reshape(-1)
    mm = mask.reshape(-1)
    lab2 = jnp.concatenate([rl, al])
    pre2 = jnp.concatenate([rp, ap])

    cnt1, sum1, h2 = _fused(lab2, pre2, mm)
    cnt1 = cnt1.reshape(NC, NS, NBINH)
    sum1 = sum1.reshape(NC, NS, NBINH)
    h2 = h2.reshape(NC, NS, 2, NBINH)

    cnt_r = jnp.sum(cnt1[0, :, :NBIN], axis=0)
    sum_r = jnp.sum(sum1[0, :, :NBIN], axis=0)
    cnt_a = jnp.sum(cnt1[1, :, :NBIN], axis=0)
    sum_a = jnp.sum(sum1[1, :, :NBIN], axis=0)
    P_r = jnp.sum(cnt1[0, :, NBIN])
    possum_r = jnp.sum(sum1[0, :, NBIN])
    P_a = jnp.sum(cnt1[1, :, NBIN])
    possum_a = jnp.sum(sum1[1, :, NBIN])
    negsum_r = jnp.sum(sum_r)
    negsum_a = jnp.sum(sum_a)

    k_r = jnp.floor(3.0 * P_r)
    k_a = jnp.floor(3.0 * P_a)
    b1r, k2r, above1_r = _select(cnt_r, sum_r, k_r)
    b1a, k2a, above1_a = _select(cnt_a, sum_a, k_a)

    b2r, kremr, above2_r = _select(jnp.sum(h2[0, :, 0, :NBIN], axis=0),
                                   jnp.sum(h2[0, :, 1, :NBIN], axis=0), k2r)
    b2a, krema, above2_a = _select(jnp.sum(h2[1, :, 0, :NBIN], axis=0),
                                   jnp.sum(h2[1, :, 1, :NBIN], axis=0), k2a)
    t_r = lax.bitcast_convert_type((b1r << 21) | (b2r << 11), jnp.float32)
    t_a = lax.bitcast_convert_type((b1a << 21) | (b2a << 11), jnp.float32)
    topk_r = above1_r + above2_r + kremr * t_r
    topk_a = above1_a + above2_a + krema * t_a

    total = jnp.float32(TOTAL)
    N_r = total - P_r
    N_a = total - P_a
    loss_r = possum_r / P_r + jnp.where(
        N_r < 3.0 * P_r, negsum_r / N_r, topk_r / (P_r * 3.0))
    loss_a = possum_a / P_a + jnp.where(
        N_a < 3.0 * P_a, negsum_a / N_a, topk_a / (P_a * 3.0))
    return loss_r + loss_a


# speedup vs baseline: 1.4159x; 1.4159x over previous
import jax
import jax.numpy as jnp
from jax import lax
from jax.experimental import pallas as pl
from jax.experimental.pallas import tpu as pltpu
from jax.experimental.pallas import tpu_sc as plsc

NC, NS, L = 2, 16, 16
TOTAL = 8 * 384 * 384
PER_T = TOTAL // NS
CH = 2048
NCH = PER_T // CH
NBIN = 1024
NBINH = 1040
HISTW = L * NBINH
U = 8

_mesh = plsc.VectorSubcoreMesh(
    core_axis_name="c", subcore_axis_name="s", num_cores=NC, num_subcores=NS)
_params = pltpu.CompilerParams(needs_layout_passes=False)


def _fused_body(rl, al, rp, ap, mm, h1, h2, out,
                st_lab0, st_lab1, st_pre0, st_pre1, st_msk0, st_msk1,
                vres, h_cnt, h_sum, red, redc, reds, comb0, comb1,
                semi0, semi1):
    cid = lax.axis_index("c")
    sid = lax.axis_index("s")
    ti = cid * NS + sid
    mbase = sid * PER_T
    lane_base = lax.iota(jnp.int32, L) * NBINH
    lane_iota = lax.iota(jnp.int32, L)
    zeros = jnp.zeros((L,), jnp.float32)
    ones = jnp.ones((L,), jnp.float32)
    posbin = jnp.full((L,), NBIN, jnp.int32)
    semi = (semi0, semi1)
    st_lab = (st_lab0, st_lab1)
    st_pre = (st_pre0, st_pre1)
    st_msk = (st_msk0, st_msk1)
    combs = (comb0, comb1)

    def zero_hists():
        def zb(j, _):
            for u in range(8):
                h_cnt[pl.ds(j * 8 * L + u * L, L)] = zeros
                h_sum[pl.ds(j * 8 * L + u * L, L)] = zeros
            return 0
        lax.fori_loop(0, HISTW // (8 * L), zb, 0)

    def zero_red2():
        def zg(g, _):
            redc[pl.ds(g * L, L)] = zeros
            reds[pl.ds(g * L, L)] = zeros
            return 0
        lax.fori_loop(0, NBINH // L, zg, 0)

    def reduce_hist(h):
        def rb(c, _):
            acc = zeros
            for l in range(L):
                acc = acc + h[pl.ds(l * NBINH + c * L, L)]
            red[pl.ds(c * L, L)] = acc
            return 0
        lax.fori_loop(0, NBINH // L, rb, 0)

    def in_select(kk):
        def tb(g, carry):
            tc, ts = carry
            return (tc + jnp.sum(redc[pl.ds(g * L, L)]),
                    ts + jnp.sum(reds[pl.ds(g * L, L)]))
        total_c, total_s = lax.fori_loop(
            0, NBIN // L, tb, (jnp.float32(0.0), jnp.float32(0.0)))

        def sb(g, carry):
            cc, cs, bv, kremv, abovev, fnd = carry
            grp_c = redc[pl.ds(g * L, L)]
            grp_s = reds[pl.ds(g * L, L)]
            cum_c = plsc.cumsum(grp_c)
            cum_s = plsc.cumsum(grp_s)
            above_c = total_c - (cc + cum_c)
            above_s = total_s - (cs + cum_s)
            hit = above_c < kk
            popc = plsc.all_reduce_population_count(hit)
            j0 = L - popc
            lmask = lane_iota == j0
            ac_at = jnp.sum(jnp.where(lmask, above_c, zeros))
            as_at = jnp.sum(jnp.where(lmask, above_s, zeros))
            first = jnp.logical_and(jnp.logical_not(fnd), popc > 0)
            bv = jnp.where(first, g * L + j0, bv)
            kremv = jnp.where(first, kk - ac_at, kremv)
            abovev = jnp.where(first, jnp.full((L,), as_at), abovev)
            fnd = jnp.logical_or(fnd, popc > 0)
            return (cc + jnp.sum(grp_c), cs + jnp.sum(grp_s),
                    bv, kremv, abovev, fnd)

        _, _, bv, kremv, abovev, _ = lax.fori_loop(
            0, NBIN // L, sb,
            (jnp.float32(0.0), jnp.float32(0.0), jnp.zeros((L,), jnp.int32),
             zeros, zeros, jnp.zeros((L,), jnp.bool_)))
        return bv, kremv, abovev, total_c, total_s

    zero_hists()
    in_h = {}

    def issue_in(c):
        buf = c & 1
        off = mbase + c * CH

        @pl.when(cid == 0)
        def _():
            pltpu.async_copy(rl.at[pl.ds(off, CH)], st_lab[buf], semi[buf])
            pltpu.async_copy(rp.at[pl.ds(off, CH)], st_pre[buf], semi[buf])

        @pl.when(cid != 0)
        def _():
            pltpu.async_copy(al.at[pl.ds(off, CH)], st_lab[buf], semi[buf])
            pltpu.async_copy(ap.at[pl.ds(off, CH)], st_pre[buf], semi[buf])

        pltpu.async_copy(mm.at[pl.ds(off, CH)], st_msk[buf], semi[buf])

    def wait_in(c):
        buf = c & 1
        off = mbase + c * CH
        pltpu.make_async_copy(rl.at[pl.ds(off, CH)], st_lab[buf], semi[buf]).wait()
        pltpu.make_async_copy(rp.at[pl.ds(off, CH)], st_pre[buf], semi[buf]).wait()
        pltpu.make_async_copy(mm.at[pl.ds(off, CH)], st_msk[buf], semi[buf]).wait()

    issue_in(0)
    for c in range(NCH):
        buf = c & 1
        if c + 1 < NCH:
            issue_in(c + 1)
        wait_in(c)
        labr = st_lab[buf]
        prer = st_pre[buf]
        mskr = st_msk[buf]

        def vec(i, _):
            ss = [pl.ds(i * U * L + u * L, L) for u in range(U)]
            vs = [pl.ds(i * U * L + u * L + c * CH, L) for u in range(U)]
            labs = [labr[s] for s in ss]
            pres = [prer[s] for s in ss]
            msks = [mskr[s] for s in ss]
            dds = [pres[u] - labs[u] for u in range(U)]
            sqs = [dds[u] * dds[u] for u in range(U)]
            plss = [sqs[u] * msks[u] for u in range(U)]
            poss = [labs[u] > 0.1 for u in range(U)]
            vvs = [jnp.where(poss[u], zeros, plss[u]) for u in range(U)]
            bits = [plsc.bitcast(vvs[u], jnp.int32) >> 21 for u in range(U)]
            addrs = [lane_base + jnp.where(poss[u], posbin, bits[u])
                     for u in range(U)]
            for u in range(U):
                vres[vs[u]] = vvs[u]
            for u in range(U):
                plsc.addupdate_scatter(h_sum, [addrs[u]], plss[u])
                plsc.addupdate_scatter(h_cnt, [addrs[u]], ones)
            return 0

        lax.fori_loop(0, CH // (U * L), vec, 0)

    reduce_hist(h_cnt)
    pltpu.sync_copy(red, h1.at[pl.ds(ti * 2 * NBINH, NBINH)])
    reduce_hist(h_sum)
    pltpu.sync_copy(red, h1.at[pl.ds(ti * 2 * NBINH + NBINH, NBINH)])

    zero_hists()
    zero_red2()
    plsc.subcore_barrier()

    def readback(src_hbm):
        cb = cid * (NS * 2 * NBINH)
        rb_h = {}

        def issue_rb(j):
            rb_h[j] = pltpu.async_copy(
                src_hbm.at[pl.ds(cb + j * 2 * NBINH, 2 * NBINH)],
                combs[j & 1], semi[j & 1])

        issue_rb(0)
        for j in range(NS):
            if j + 1 < NS:
                issue_rb(j + 1)
            rb_h.pop(j).wait()
            cmb = combs[j & 1]

            def rbody(g, _):
                s = pl.ds(g * L, L)
                redc[s] = redc[s] + cmb[pl.ds(g * L, L)]
                reds[s] = reds[s] + cmb[pl.ds(g * L + NBINH, L)]
                return 0
            lax.fori_loop(0, NBINH // L, rbody, 0)

    readback(h1)

    Pv = jnp.sum(redc[pl.ds(NBIN, L)])
    possum = jnp.sum(reds[pl.ds(NBIN, L)])
    kk1 = jnp.full((L,), 3.0 * Pv)
    b1v, krem1, above1, N_c, negsum = in_select(kk1)

    def vec2(i, _):
        ss = [pl.ds(i * U * L + u * L, L) for u in range(U)]
        vvs = [vres[s] for s in ss]
        bits = [plsc.bitcast(vvs[u], jnp.int32) for u in range(U)]
        matches = [(bits[u] >> 21) == b1v for u in range(U)]
        addrs = [lane_base + ((bits[u] >> 11) & 0x3FF) for u in range(U)]
        for u in range(U):
            plsc.addupdate_scatter(h_sum, [addrs[u]], vvs[u], mask=matches[u])
            plsc.addupdate_scatter(h_cnt, [addrs[u]], ones, mask=matches[u])
        return 0

    lax.fori_loop(0, PER_T // (U * L), vec2, 0)

    reduce_hist(h_cnt)
    pltpu.sync_copy(red, h2.at[pl.ds(ti * 2 * NBINH, NBINH)])
    reduce_hist(h_sum)
    pltpu.sync_copy(red, h2.at[pl.ds(ti * 2 * NBINH + NBINH, NBINH)])

    zero_red2()
    plsc.subcore_barrier()
    readback(h2)

    b2v, krem2, above2, _, _ = in_select(krem1)

    t_lo = plsc.bitcast((b1v << 21) | (b2v << 11), jnp.float32)
    topk = above1 + above2 + krem2 * t_lo
    Pvv = jnp.full((L,), Pv)
    possumv = jnp.full((L,), possum)
    negsumv = jnp.full((L,), negsum)
    N_cv = jnp.full((L,), N_c)
    pos_loss = possumv / Pvv
    neg_loss = jnp.where(N_cv < 3.0 * Pvv, negsumv / N_cv, topk / (Pvv * 3.0))
    loss = pos_loss + neg_loss

    @pl.when(sid == 0)
    def _():
        red[pl.ds(0, L)] = loss
        pltpu.sync_copy(red.at[pl.ds(0, L)], out.at[pl.ds(cid * L, L)])


_fused = pl.kernel(
    _fused_body,
    out_type=[
        jax.ShapeDtypeStruct((NC * NS * 2 * NBINH,), jnp.float32),
        jax.ShapeDtypeStruct((NC * NS * 2 * NBINH,), jnp.float32),
        jax.ShapeDtypeStruct((NC * L,), jnp.float32),
    ],
    mesh=_mesh,
    compiler_params=_params,
    scratch_types=[
        pltpu.VMEM((CH,), jnp.float32),
        pltpu.VMEM((CH,), jnp.float32),
        pltpu.VMEM((CH,), jnp.float32),
        pltpu.VMEM((CH,), jnp.float32),
        pltpu.VMEM((CH,), jnp.float32),
        pltpu.VMEM((CH,), jnp.float32),
        pltpu.VMEM((PER_T,), jnp.float32),
        pltpu.VMEM((HISTW,), jnp.float32),
        pltpu.VMEM((HISTW,), jnp.float32),
        pltpu.VMEM((NBINH,), jnp.float32),
        pltpu.VMEM((NBINH,), jnp.float32),
        pltpu.VMEM((NBINH,), jnp.float32),
        pltpu.VMEM((2 * NBINH,), jnp.float32),
        pltpu.VMEM((2 * NBINH,), jnp.float32),
        pltpu.SemaphoreType.DMA,
        pltpu.SemaphoreType.DMA,
    ],
)


def kernel(region_scores_label, affinity_socres_label, region_scores_pre,
           affinity_scores_pre, mask):
    rl = region_scores_label.reshape(-1)
    al = affinity_socres_label.reshape(-1)
    rp = region_scores_pre.reshape(-1)
    ap = affinity_scores_pre.reshape(-1)
    mm = mask.reshape(-1)
    _, _, out = _fused(rl, al, rp, ap, mm)
    return out[0] + out[L]


# speedup vs baseline: 1.6213x; 1.1450x over previous
import jax
import jax.numpy as jnp
from jax import lax
from jax.experimental import pallas as pl
from jax.experimental.pallas import tpu as pltpu
from jax.experimental.pallas import tpu_sc as plsc

NC, NS, L = 2, 16, 16
TOTAL = 8 * 384 * 384
PER_T = TOTAL // NS
CH = 4096
NCH = PER_T // CH
NBIN = 512
NBINH = 528
HISTW = L * NBINH
U = 8

_mesh = plsc.VectorSubcoreMesh(
    core_axis_name="c", subcore_axis_name="s", num_cores=NC, num_subcores=NS)
_params = pltpu.CompilerParams(needs_layout_passes=False)


def _fused_body(rl, al, rp, ap, mm, h1, h2, out,
                st_lab0, st_lab1, st_pre0, st_pre1, st_msk0, st_msk1,
                vres, h_cnt, h_sum, red, redc, reds, comb0, comb1,
                semi0, semi1):
    cid = lax.axis_index("c")
    sid = lax.axis_index("s")
    ti = cid * NS + sid
    mbase = sid * PER_T
    lane_base = lax.iota(jnp.int32, L) * NBINH
    lane_iota = lax.iota(jnp.int32, L)
    zeros = jnp.zeros((L,), jnp.float32)
    ones = jnp.ones((L,), jnp.float32)
    posbin = jnp.full((L,), NBIN, jnp.int32)
    semi = (semi0, semi1)
    st_lab = (st_lab0, st_lab1)
    st_pre = (st_pre0, st_pre1)
    st_msk = (st_msk0, st_msk1)
    combs = (comb0, comb1)

    def zero_hists():
        def zb(j, _):
            for u in range(8):
                h_cnt[pl.ds(j * 8 * L + u * L, L)] = zeros
                h_sum[pl.ds(j * 8 * L + u * L, L)] = zeros
            return 0
        lax.fori_loop(0, HISTW // (8 * L), zb, 0)

    def zero_red2():
        def zg(g, _):
            redc[pl.ds(g * L, L)] = zeros
            reds[pl.ds(g * L, L)] = zeros
            return 0
        lax.fori_loop(0, NBINH // L, zg, 0)

    def reduce_hist(h):
        def rb(c, _):
            acc = zeros
            for l in range(L):
                acc = acc + h[pl.ds(l * NBINH + c * L, L)]
            red[pl.ds(c * L, L)] = acc
            return 0
        lax.fori_loop(0, NBINH // L, rb, 0)

    def in_select(kk):
        def sb(t, carry):
            cc, cs, bv, kremv, abovev = carry
            g = NBIN // L - 1 - t
            grp_c = redc[pl.ds(g * L, L)]
            grp_s = reds[pl.ds(g * L, L)]
            cum_c = plsc.cumsum(grp_c)
            cum_s = plsc.cumsum(grp_s)
            gsum_c = jnp.sum(grp_c)
            gsum_s = jnp.sum(grp_s)
            above_c = cc + (gsum_c - cum_c)
            above_s = cs + (gsum_s - cum_s)
            hit = above_c < kk
            popc = plsc.all_reduce_population_count(hit)
            j0 = L - popc
            lmask = lane_iota == j0
            ac_at = jnp.sum(jnp.where(lmask, above_c, zeros))
            as_at = jnp.sum(jnp.where(lmask, above_s, zeros))
            any_ = popc > 0
            bv = jnp.where(any_, g * L + j0, bv)
            kremv = jnp.where(any_, kk - ac_at, kremv)
            abovev = jnp.where(any_, jnp.full((L,), as_at), abovev)
            return (cc + gsum_c, cs + gsum_s, bv, kremv, abovev)

        total_c, total_s, bv, kremv, abovev = lax.fori_loop(
            0, NBIN // L, sb,
            (jnp.float32(0.0), jnp.float32(0.0), jnp.zeros((L,), jnp.int32),
             zeros, zeros))
        return bv, kremv, abovev, total_c, total_s

    zero_hists()
    in_h = {}

    def issue_in(c):
        buf = c & 1
        off = mbase + c * CH

        @pl.when(cid == 0)
        def _():
            pltpu.async_copy(rl.at[pl.ds(off, CH)], st_lab[buf], semi[buf])
            pltpu.async_copy(rp.at[pl.ds(off, CH)], st_pre[buf], semi[buf])

        @pl.when(cid != 0)
        def _():
            pltpu.async_copy(al.at[pl.ds(off, CH)], st_lab[buf], semi[buf])
            pltpu.async_copy(ap.at[pl.ds(off, CH)], st_pre[buf], semi[buf])

        pltpu.async_copy(mm.at[pl.ds(off, CH)], st_msk[buf], semi[buf])

    def wait_in(c):
        buf = c & 1
        off = mbase + c * CH
        pltpu.make_async_copy(rl.at[pl.ds(off, CH)], st_lab[buf], semi[buf]).wait()
        pltpu.make_async_copy(rp.at[pl.ds(off, CH)], st_pre[buf], semi[buf]).wait()
        pltpu.make_async_copy(mm.at[pl.ds(off, CH)], st_msk[buf], semi[buf]).wait()

    issue_in(0)
    for c in range(NCH):
        buf = c & 1
        if c + 1 < NCH:
            issue_in(c + 1)
        wait_in(c)
        labr = st_lab[buf]
        prer = st_pre[buf]
        mskr = st_msk[buf]

        def vec(i, _):
            ss = [pl.ds(i * U * L + u * L, L) for u in range(U)]
            vs = [pl.ds(i * U * L + u * L + c * CH, L) for u in range(U)]
            labs = [labr[s] for s in ss]
            pres = [prer[s] for s in ss]
            msks = [mskr[s] for s in ss]
            dds = [pres[u] - labs[u] for u in range(U)]
            sqs = [dds[u] * dds[u] for u in range(U)]
            plss = [sqs[u] * msks[u] for u in range(U)]
            poss = [labs[u] > 0.1 for u in range(U)]
            vvs = [jnp.where(poss[u], zeros, plss[u]) for u in range(U)]
            bits = [plsc.bitcast(vvs[u], jnp.int32) >> 21 for u in range(U)]
            addrs = [lane_base + jnp.where(poss[u], posbin, bits[u])
                     for u in range(U)]
            for u in range(U):
                vres[vs[u]] = vvs[u]
            for u in range(U):
                plsc.addupdate_scatter(h_sum, [addrs[u]], plss[u])
                plsc.addupdate_scatter(h_cnt, [addrs[u]], ones)
            return 0

        lax.fori_loop(0, CH // (U * L), vec, 0)

    reduce_hist(h_cnt)
    pltpu.sync_copy(red, h1.at[pl.ds(ti * 2 * NBINH, NBINH)])
    reduce_hist(h_sum)
    pltpu.sync_copy(red, h1.at[pl.ds(ti * 2 * NBINH + NBINH, NBINH)])

    zero_hists()
    zero_red2()
    plsc.subcore_barrier()

    def readback(src_hbm):
        cb = cid * (NS * 2 * NBINH)
        rb_h = {}

        def issue_rb(j):
            rb_h[j] = pltpu.async_copy(
                src_hbm.at[pl.ds(cb + j * 2 * NBINH, 2 * NBINH)],
                combs[j & 1], semi[j & 1])

        issue_rb(0)
        for j in range(NS):
            if j + 1 < NS:
                issue_rb(j + 1)
            rb_h.pop(j).wait()
            cmb = combs[j & 1]

            def rbody(g, _):
                s = pl.ds(g * L, L)
                redc[s] = redc[s] + cmb[pl.ds(g * L, L)]
                reds[s] = reds[s] + cmb[pl.ds(g * L + NBINH, L)]
                return 0
            lax.fori_loop(0, NBINH // L, rbody, 0)

    readback(h1)

    Pv = jnp.sum(redc[pl.ds(NBIN, L)])
    possum = jnp.sum(reds[pl.ds(NBIN, L)])
    kk1 = jnp.full((L,), 3.0 * Pv)
    b1v, krem1, above1, N_c, negsum = in_select(kk1)

    def vec2(i, _):
        ss = [pl.ds(i * U * L + u * L, L) for u in range(U)]
        vvs = [vres[s] for s in ss]
        bits = [plsc.bitcast(vvs[u], jnp.int32) for u in range(U)]
        matches = [(bits[u] >> 21) == b1v for u in range(U)]
        addrs = [lane_base + ((bits[u] >> 12) & 0x1FF) for u in range(U)]
        for u in range(U):
            plsc.addupdate_scatter(h_sum, [addrs[u]], vvs[u], mask=matches[u])
            plsc.addupdate_scatter(h_cnt, [addrs[u]], ones, mask=matches[u])
        return 0

    lax.fori_loop(0, PER_T // (U * L), vec2, 0)

    reduce_hist(h_cnt)
    pltpu.sync_copy(red, h2.at[pl.ds(ti * 2 * NBINH, NBINH)])
    reduce_hist(h_sum)
    pltpu.sync_copy(red, h2.at[pl.ds(ti * 2 * NBINH + NBINH, NBINH)])

    zero_red2()
    plsc.subcore_barrier()
    readback(h2)

    b2v, krem2, above2, _, _ = in_select(krem1)

    t_lo = plsc.bitcast((b1v << 21) | (b2v << 12), jnp.float32)
    topk = above1 + above2 + krem2 * t_lo
    Pvv = jnp.full((L,), Pv)
    possumv = jnp.full((L,), possum)
    negsumv = jnp.full((L,), negsum)
    N_cv = jnp.full((L,), N_c)
    pos_loss = possumv / Pvv
    neg_loss = jnp.where(N_cv < 3.0 * Pvv, negsumv / N_cv, topk / (Pvv * 3.0))
    loss = pos_loss + neg_loss

    @pl.when(sid == 0)
    def _():
        red[pl.ds(0, L)] = loss
        pltpu.sync_copy(red.at[pl.ds(0, L)], out.at[pl.ds(cid * L, L)])


_fused = pl.kernel(
    _fused_body,
    out_type=[
        jax.ShapeDtypeStruct((NC * NS * 2 * NBINH,), jnp.float32),
        jax.ShapeDtypeStruct((NC * NS * 2 * NBINH,), jnp.float32),
        jax.ShapeDtypeStruct((NC * L,), jnp.float32),
    ],
    mesh=_mesh,
    compiler_params=_params,
    scratch_types=[
        pltpu.VMEM((CH,), jnp.float32),
        pltpu.VMEM((CH,), jnp.float32),
        pltpu.VMEM((CH,), jnp.float32),
        pltpu.VMEM((CH,), jnp.float32),
        pltpu.VMEM((CH,), jnp.float32),
        pltpu.VMEM((CH,), jnp.float32),
        pltpu.VMEM((PER_T,), jnp.float32),
        pltpu.VMEM((HISTW,), jnp.float32),
        pltpu.VMEM((HISTW,), jnp.float32),
        pltpu.VMEM((NBINH,), jnp.float32),
        pltpu.VMEM((NBINH,), jnp.float32),
        pltpu.VMEM((NBINH,), jnp.float32),
        pltpu.VMEM((2 * NBINH,), jnp.float32),
        pltpu.VMEM((2 * NBINH,), jnp.float32),
        pltpu.SemaphoreType.DMA,
        pltpu.SemaphoreType.DMA,
    ],
)


def kernel(region_scores_label, affinity_socres_label, region_scores_pre,
           affinity_scores_pre, mask):
    rl = region_scores_label.reshape(-1)
    al = affinity_socres_label.reshape(-1)
    rp = region_scores_pre.reshape(-1)
    ap = affinity_scores_pre.reshape(-1)
    mm = mask.reshape(-1)
    _, _, out = _fused(rl, al, rp, ap, mm)
    return out[0] + out[L]
